# baseline (speedup 1.0000x reference)
import jax
import jax.numpy as jnp
from jax.experimental import pallas as pl

_PRE_NMS_THRESH = 0.05
_PRE_NMS_TOP_N = 1000
_NMS_THRESH = 0.6
_POST_TOP_N = 100
_PAD = 1024


def _decode_nms_kernel(reg_ref, locx_ref, locy_ref, lab_ref, ts_ref,
                       wid_ref, hgt_ref, boxes_ref, keep_ref, score_ref):
    l = reg_ref[:, 0, :]
    t = reg_ref[:, 1, :]
    r = reg_ref[:, 2, :]
    b = reg_ref[:, 3, :]
    x = locx_ref[...]
    y = locy_ref[...]
    wid = wid_ref[...]
    hgt = hgt_ref[...]

    x1 = jnp.minimum(jnp.maximum(x - l, 0.0), wid - 1.0)
    y1 = jnp.minimum(jnp.maximum(y - t, 0.0), hgt - 1.0)
    x2 = jnp.minimum(jnp.maximum(x + r, 0.0), wid - 1.0)
    y2 = jnp.minimum(jnp.maximum(y + b, 0.0), hgt - 1.0)
    boxes_ref[:, 0, :] = x1
    boxes_ref[:, 1, :] = y1
    boxes_ref[:, 2, :] = x2
    boxes_ref[:, 3, :] = y2

    ts = ts_ref[...]
    valid = (ts > 0.0) & ((x2 - x1) >= 0.0) & ((y2 - y1) >= 0.0)
    score_ref[...] = jnp.where(valid, jnp.sqrt(jnp.maximum(ts, 1e-12)), 0.0)

    off = lab_ref[...] * (jnp.maximum(wid, hgt) + 1.0)
    ox1 = x1 + off
    oy1 = y1 + off
    ox2 = x2 + off
    oy2 = y2 + off
    area = jnp.maximum(ox2 - ox1 + 1.0, 0.0) * jnp.maximum(oy2 - oy1 + 1.0, 0.0)
    idx = jax.lax.broadcasted_iota(jnp.int32, ox1.shape, 1)

    def body(i, keep):
        oh = (idx == i).astype(jnp.float32)
        bx1 = jnp.sum(ox1 * oh, axis=1, keepdims=True)
        by1 = jnp.sum(oy1 * oh, axis=1, keepdims=True)
        bx2 = jnp.sum(ox2 * oh, axis=1, keepdims=True)
        by2 = jnp.sum(oy2 * oh, axis=1, keepdims=True)
        barea = jnp.sum(area * oh, axis=1, keepdims=True)
        keep_i = jnp.sum(keep * oh, axis=1, keepdims=True)
        ix1 = jnp.maximum(bx1, ox1)
        iy1 = jnp.maximum(by1, oy1)
        ix2 = jnp.minimum(bx2, ox2)
        iy2 = jnp.minimum(by2, oy2)
        inter = jnp.maximum(ix2 - ix1 + 1.0, 0.0) * jnp.maximum(iy2 - iy1 + 1.0, 0.0)
        union = barea + area - inter
        iou = inter / jnp.maximum(union, 1e-6)
        sup = (iou > _NMS_THRESH) & (idx > i) & (keep_i > 0.0)
        return jnp.where(sup, 0.0, keep)

    keep0 = valid.astype(jnp.float32)
    keep_ref[...] = jax.lax.fori_loop(0, _PAD, body, keep0)


def kernel(locations, box_cls, box_regression, centerness, image_sizes):
    N, C, H, W = box_cls.shape
    HW = H * W
    cls = jax.nn.sigmoid(jnp.transpose(box_cls, (0, 2, 3, 1)).reshape(N, -1, C))
    cent = jax.nn.sigmoid(jnp.transpose(centerness, (0, 2, 3, 1)).reshape(N, -1))
    candidate = cls > _PRE_NMS_THRESH
    scores = cls * cent[:, :, None]
    flat = jnp.where(candidate, scores, 0.0).reshape(N, -1)
    ts, ti = jax.lax.top_k(flat, _PRE_NMS_TOP_N)
    loc_idx = ti // C
    labels = ti % C + 1

    reg = jnp.transpose(box_regression, (0, 2, 3, 1)).reshape(N, HW, 4)
    reg_sel = jnp.take_along_axis(reg, loc_idx[..., None], axis=1)
    locx = locations[:, 0][loc_idx]
    locy = locations[:, 1][loc_idx]

    pad = _PAD - _PRE_NMS_TOP_N

    def padl(a):
        return jnp.pad(a, ((0, 0), (0, pad)))

    reg_p = jnp.pad(reg_sel.transpose(0, 2, 1), ((0, 0), (0, 0), (0, pad)))
    locx_p = padl(locx)
    locy_p = padl(locy)
    lab_p = padl(labels.astype(jnp.float32))
    ts_p = padl(ts)
    wid_b = jnp.broadcast_to(
        image_sizes[:, 1].astype(jnp.float32)[:, None], (N, _PAD))
    hgt_b = jnp.broadcast_to(
        image_sizes[:, 0].astype(jnp.float32)[:, None], (N, _PAD))

    boxes, keep, sc = pl.pallas_call(
        _decode_nms_kernel,
        out_shape=(jax.ShapeDtypeStruct((N, 4, _PAD), jnp.float32),
                   jax.ShapeDtypeStruct((N, _PAD), jnp.float32),
                   jax.ShapeDtypeStruct((N, _PAD), jnp.float32)),
    )(reg_p, locx_p, locy_p, lab_p, ts_p, wid_b, hgt_b)

    masked = keep * sc
    fs, fidx = jax.lax.top_k(masked, _POST_TOP_N)
    fboxes = jnp.take_along_axis(
        boxes.transpose(0, 2, 1), fidx[..., None], axis=1)
    flabels = jnp.take_along_axis(padl(labels), fidx, axis=1)
    return fboxes, fs, flabels
